# baseline (device time: 19540 ns/iter reference)
import os

import jax
import jax.numpy as jnp
from jax import lax
from jax.experimental import pallas as pl
from jax.experimental.pallas import tpu as pltpu

_NO_COMM = os.environ.get("KERNEL_NO_COMM", "0") == "1"

N_DEV = 4
B, SQ, SKV = 2, 256, 256
HQ, DH = 16, 64
H_LOC = HQ // N_DEV
C_LOC = H_LOC * DH
D_MODEL = 512
BLK = 64
SCALE = 0.125

_MESH = pl.DeviceIdType.MESH


def kernel(x, Wq, K_ext, V_ext, Wo):
    def body(x_ref, wq_ref, k_ref, v_ref, wo_ref, out_ref,
             ctx_ref, comm_ref, wq_v, k_v, v_v, wo_v, acc_ref,
             dma_sems, send_sems, recv_sems):
        my = lax.axis_index("i")
        left = lax.rem(my + N_DEV - 1, N_DEV)
        right = lax.rem(my + 1, N_DEV)
        diag = lax.rem(my + 2, N_DEV)

        if not _NO_COMM:
            bar = pltpu.get_barrier_semaphore()
            for p in (left, right, diag):
                pl.semaphore_signal(bar, inc=1, device_id=(p,),
                                    device_id_type=_MESH)

        wq_dma = pltpu.make_async_copy(
            wq_ref.at[:, pl.ds(my * C_LOC, C_LOC)], wq_v, dma_sems.at[0])
        k_dma = pltpu.make_async_copy(k_ref, k_v, dma_sems.at[1])
        v_dma = pltpu.make_async_copy(v_ref, v_v, dma_sems.at[2])
        wo_dma = pltpu.make_async_copy(wo_ref, wo_v, dma_sems.at[3])
        wq_dma.start()
        k_dma.start()
        v_dma.start()
        wo_dma.start()

        qb = lax.broadcasted_iota(jnp.int32, (SQ, SKV), 0) // BLK
        kb = lax.broadcasted_iota(jnp.int32, (SQ, SKV), 1) // BLK
        bias = jnp.where(kb <= qb, 0.0, -1e9).astype(jnp.float32)

        wq_dma.wait()
        k_dma.wait()
        v_dma.wait()

        wq_my = wq_v[...].astype(jnp.bfloat16)
        for b in range(B):
            x_b = x_ref[b].astype(jnp.bfloat16)
            q_all = jnp.dot(x_b, wq_my,
                            preferred_element_type=jnp.float32)
            q_all = q_all.astype(jnp.bfloat16)
            for h in range(H_LOC):
                qh = q_all[:, h * DH:(h + 1) * DH]
                kh = k_v[b, :, h, :].astype(jnp.bfloat16)
                s = lax.dot_general(
                    qh, kh, (((1,), (1,)), ((), ())),
                    preferred_element_type=jnp.float32)
                w = jnp.exp(s * SCALE + bias)
                denom = jnp.sum(w, axis=1, keepdims=True)
                vh = v_v[b, :, h, :].astype(jnp.bfloat16)
                num = jnp.dot(w.astype(jnp.bfloat16), vh,
                              preferred_element_type=jnp.float32)
                ctx_h = num * (1.0 / denom)
                ctx_ref[b, :, h * DH:(h + 1) * DH] = ctx_h.astype(jnp.bfloat16)

        wo_dma.wait()

        if _NO_COMM:
            for b in range(B):
                out_ref[b] = jnp.dot(
                    ctx_ref[b],
                    wo_v[pl.ds(my * C_LOC, C_LOC), :].astype(jnp.bfloat16),
                    preferred_element_type=jnp.float32).astype(jnp.bfloat16)
            return

        bar_wait = pl.semaphore_wait(bar, 3)

        sends = []
        for dst_dev, slot in ((diag, 2), (right, 0), (left, 1)):
            r = pltpu.make_async_remote_copy(
                src_ref=ctx_ref,
                dst_ref=comm_ref.at[slot],
                send_sem=send_sems.at[slot],
                recv_sem=recv_sems.at[slot],
                device_id=(dst_dev,),
                device_id_type=_MESH,
            )
            r.start()
            sends.append(r)

        wo_my = wo_v[pl.ds(my * C_LOC, C_LOC), :].astype(jnp.bfloat16)
        for b in range(B):
            acc_ref[b] = jnp.dot(
                ctx_ref[b], wo_my,
                preferred_element_type=jnp.float32)

        for origin, slot in ((left, 0), (right, 1), (diag, 2)):
            recv = pltpu.make_async_remote_copy(
                src_ref=ctx_ref,
                dst_ref=comm_ref.at[slot],
                send_sem=send_sems.at[slot],
                recv_sem=recv_sems.at[slot],
                device_id=(origin,),
                device_id_type=_MESH,
            )
            recv.wait_recv()
            wo_o = wo_v[pl.ds(origin * C_LOC, C_LOC), :].astype(jnp.bfloat16)
            if slot < 2:
                for b in range(B):
                    acc_ref[b] = acc_ref[b] + jnp.dot(
                        comm_ref[slot, b], wo_o,
                        preferred_element_type=jnp.float32)
            else:
                for b in range(B):
                    out_ref[b] = (acc_ref[b] + jnp.dot(
                        comm_ref[slot, b], wo_o,
                        preferred_element_type=jnp.float32)
                    ).astype(jnp.bfloat16)

        for r in sends:
            r.wait_send()

    return pl.pallas_call(
        body,
        out_shape=jax.ShapeDtypeStruct((B, SQ, D_MODEL), jnp.bfloat16),
        in_specs=(
            [pl.BlockSpec(memory_space=pltpu.MemorySpace.VMEM)]
            + [pl.BlockSpec(memory_space=pltpu.MemorySpace.HBM)] * 4
        ),
        out_specs=pl.BlockSpec(memory_space=pltpu.MemorySpace.VMEM),
        scratch_shapes=[
            pltpu.VMEM((B, SQ, C_LOC), jnp.bfloat16),
            pltpu.VMEM((3, B, SQ, C_LOC), jnp.bfloat16),
            pltpu.VMEM((D_MODEL, C_LOC), jnp.float32),
            pltpu.VMEM((B, SKV, H_LOC, DH), jnp.float32),
            pltpu.VMEM((B, SKV, H_LOC, DH), jnp.float32),
            pltpu.VMEM((HQ * DH, D_MODEL), jnp.float32),
            pltpu.VMEM((B, SQ, D_MODEL), jnp.float32),
            pltpu.SemaphoreType.DMA((4,)),
            pltpu.SemaphoreType.DMA((3,)),
            pltpu.SemaphoreType.DMA((3,)),
        ],
        compiler_params=(None if _NO_COMM
                         else pltpu.CompilerParams(collective_id=0)),
    )(x, Wq, K_ext, V_ext, Wo)


# device time: 16033 ns/iter; 1.2187x vs baseline; 1.2187x over previous
import os

import jax
import jax.numpy as jnp
from jax import lax
from jax.experimental import pallas as pl
from jax.experimental.pallas import tpu as pltpu

_NO_COMM = os.environ.get("KERNEL_NO_COMM", "0") == "1"

N_DEV = 4
B, SQ, SKV = 2, 256, 256
HQ, DH = 16, 64
H_LOC = HQ // N_DEV
C_LOC = H_LOC * DH
D_MODEL = 512
BLK = 64
SCALE = 0.125

_MESH = pl.DeviceIdType.MESH


def kernel(x, Wq, K_ext, V_ext, Wo):
    my_dev = lax.axis_index("i")
    x_bf = x.astype(jnp.bfloat16)
    wq_loc = lax.dynamic_slice(
        Wq, (0, my_dev * C_LOC), (D_MODEL, C_LOC)).astype(jnp.bfloat16)
    k_t = K_ext.transpose(0, 2, 3, 1).astype(jnp.bfloat16)
    v_t = V_ext.transpose(0, 2, 3, 1).astype(jnp.bfloat16)
    wo_bf = Wo.astype(jnp.bfloat16)

    def body(x_ref, wq_ref, k_ref, v_ref, wo_ref, out_ref,
             ctx_ref, comm_ref, acc_ref, send_sems, recv_sems):
        my = lax.axis_index("i")
        left = lax.rem(my + N_DEV - 1, N_DEV)
        right = lax.rem(my + 1, N_DEV)
        diag = lax.rem(my + 2, N_DEV)

        if not _NO_COMM:
            bar = pltpu.get_barrier_semaphore()
            for p in (left, right, diag):
                pl.semaphore_signal(bar, inc=1, device_id=(p,),
                                    device_id_type=_MESH)

        qb = lax.broadcasted_iota(jnp.int32, (SQ, SKV), 0) // BLK
        kb = lax.broadcasted_iota(jnp.int32, (SQ, SKV), 1) // BLK
        bias = jnp.where(kb <= qb, 0.0, -1e9).astype(jnp.float32)

        for b in range(B):
            q_all = jnp.dot(x_ref[b], wq_ref[...],
                            preferred_element_type=jnp.float32)
            q_all = q_all.astype(jnp.bfloat16)
            for h in range(H_LOC):
                qh = q_all[:, h * DH:(h + 1) * DH]
                s = jnp.dot(qh, k_ref[b, h],
                            preferred_element_type=jnp.float32)
                w = jnp.exp(s * SCALE + bias)
                denom = jnp.sum(w, axis=1, keepdims=True)
                num = lax.dot_general(
                    w.astype(jnp.bfloat16), v_ref[b, h],
                    (((1,), (1,)), ((), ())),
                    preferred_element_type=jnp.float32)
                ctx_h = num * (1.0 / denom)
                ctx_ref[b, :, h * DH:(h + 1) * DH] = ctx_h.astype(jnp.bfloat16)

        if _NO_COMM:
            for b in range(B):
                out_ref[b] = jnp.dot(
                    ctx_ref[b], wo_ref[pl.ds(my * C_LOC, C_LOC), :],
                    preferred_element_type=jnp.float32).astype(jnp.bfloat16)
            return

        pl.semaphore_wait(bar, 3)

        sends = []
        for dst_dev, slot in ((diag, 2), (right, 0), (left, 1)):
            r = pltpu.make_async_remote_copy(
                src_ref=ctx_ref,
                dst_ref=comm_ref.at[slot],
                send_sem=send_sems.at[slot],
                recv_sem=recv_sems.at[slot],
                device_id=(dst_dev,),
                device_id_type=_MESH,
            )
            r.start()
            sends.append(r)

        wo_my = wo_ref[pl.ds(my * C_LOC, C_LOC), :]
        for b in range(B):
            acc_ref[b] = jnp.dot(
                ctx_ref[b], wo_my,
                preferred_element_type=jnp.float32)

        for origin, slot in ((left, 0), (right, 1), (diag, 2)):
            recv = pltpu.make_async_remote_copy(
                src_ref=ctx_ref,
                dst_ref=comm_ref.at[slot],
                send_sem=send_sems.at[slot],
                recv_sem=recv_sems.at[slot],
                device_id=(origin,),
                device_id_type=_MESH,
            )
            recv.wait_recv()
            wo_o = wo_ref[pl.ds(origin * C_LOC, C_LOC), :]
            if slot < 2:
                for b in range(B):
                    acc_ref[b] = acc_ref[b] + jnp.dot(
                        comm_ref[slot, b], wo_o,
                        preferred_element_type=jnp.float32)
            else:
                for b in range(B):
                    out_ref[b] = (acc_ref[b] + jnp.dot(
                        comm_ref[slot, b], wo_o,
                        preferred_element_type=jnp.float32)
                    ).astype(jnp.bfloat16)

        for r in sends:
            r.wait_send()

    return pl.pallas_call(
        body,
        out_shape=jax.ShapeDtypeStruct((B, SQ, D_MODEL), jnp.bfloat16),
        in_specs=[pl.BlockSpec(memory_space=pltpu.MemorySpace.VMEM)] * 5,
        out_specs=pl.BlockSpec(memory_space=pltpu.MemorySpace.VMEM),
        scratch_shapes=[
            pltpu.VMEM((B, SQ, C_LOC), jnp.bfloat16),
            pltpu.VMEM((3, B, SQ, C_LOC), jnp.bfloat16),
            pltpu.VMEM((B, SQ, D_MODEL), jnp.float32),
            pltpu.SemaphoreType.DMA((3,)),
            pltpu.SemaphoreType.DMA((3,)),
        ],
        compiler_params=(None if _NO_COMM
                         else pltpu.CompilerParams(collective_id=0)),
    )(x_bf, wq_loc, k_t, v_t, wo_bf)
